# baseline (speedup 1.0000x reference)
import functools

import jax
import jax.numpy as jnp
from jax import lax
from jax.experimental import pallas as pl
from jax.experimental.pallas import tpu as pltpu
from jax.experimental.pallas import tpu_sc as plsc

VOCAB = 100000
DIM = 300
DIM_I = 150
B = 1024
MAX_NORM = 1.0


@functools.lru_cache(maxsize=None)
def _make_sc_gather():
    info = plsc.get_sparse_core_info()
    nc, ns, nl = info.num_cores, info.num_subcores, info.num_lanes
    b_per_w = B // (nc * ns)

    def body(idx_hbm, table_hbm, out_hbm, idx_v, rows_v, sem):
        wid = lax.axis_index("s") * nc + lax.axis_index("c")
        base = wid * b_per_w
        pltpu.sync_copy(idx_hbm.at[pl.ds(base, b_per_w)], idx_v)
        descs = []
        for c in range(b_per_w // nl):
            vec = idx_v[pl.ds(c * nl, nl)]
            for j in range(nl):
                row = vec[j]
                i = c * nl + j
                d = pltpu.make_async_copy(
                    table_hbm.at[pl.ds(row, 1)], rows_v.at[pl.ds(i, 1)], sem
                )
                d.start()
                descs.append(d)
        for d in descs:
            d.wait()
        pltpu.sync_copy(rows_v, out_hbm.at[pl.ds(base, b_per_w)])

    return pl.kernel(
        body,
        out_type=jax.ShapeDtypeStruct((B, DIM_I), jnp.int32),
        mesh=plsc.VectorSubcoreMesh(core_axis_name="c", subcore_axis_name="s"),
        scratch_types=[
            pltpu.VMEM((b_per_w,), jnp.int32),
            pltpu.VMEM((b_per_w, DIM_I), jnp.int32),
            pltpu.SemaphoreType.DMA,
        ],
    )


V_BLK = 4096


def _mm_body(e_ref, wt_ref, b_ref, o_ref, ent_ref):
    @pl.when(pl.program_id(0) == 0)
    def _():
        e = e_ref[...].astype(jnp.float32)
        ss = jnp.sum(e * e, axis=1, keepdims=True)
        norm = jnp.sqrt(ss)
        scale = jnp.where(norm > MAX_NORM, MAX_NORM / (norm + 1e-7), 1.0)
        ent_ref[...] = (e * scale).astype(jnp.bfloat16).T

    o_ref[...] = lax.dot_general(
        wt_ref[...].astype(jnp.bfloat16), ent_ref[...],
        (((0,), (0,)), ((), ())),
        preferred_element_type=jnp.float32,
    ) + b_ref[...].reshape(V_BLK, 1)


def _tc_matmul(e, Wt, b1):
    nblk = pl.cdiv(VOCAB, V_BLK)
    return pl.pallas_call(
        _mm_body,
        grid=(nblk,),
        in_specs=[
            pl.BlockSpec((B, DIM), lambda i: (0, 0)),
            pl.BlockSpec((DIM, V_BLK), lambda i: (0, i)),
            pl.BlockSpec((V_BLK,), lambda i: (i,)),
        ],
        out_specs=pl.BlockSpec((V_BLK, B), lambda i: (i, 0)),
        out_shape=jax.ShapeDtypeStruct((VOCAB, B), jnp.float32),
        scratch_shapes=[pltpu.VMEM((DIM, B), jnp.bfloat16)],
    )(e, Wt, b1)


def kernel(inputs, emb, W, b):
    idx = inputs.astype(jnp.int32)
    table_i = lax.bitcast_convert_type(
        emb.astype(jnp.bfloat16).reshape(VOCAB, DIM_I, 2), jnp.int32)
    e_i = _make_sc_gather()(idx, table_i)
    e = lax.bitcast_convert_type(e_i, jnp.bfloat16).reshape(B, DIM)
    out_t = _tc_matmul(e, W.T, b)
    return out_t.T


# speedup vs baseline: 2.4293x; 2.4293x over previous
import functools

import jax
import jax.numpy as jnp
from jax import lax
from jax.experimental import pallas as pl
from jax.experimental.pallas import tpu as pltpu
from jax.experimental.pallas import tpu_sc as plsc

VOCAB = 100000
DIM = 300
DIM_I = 150
B = 1024
MAX_NORM = 1.0


@functools.lru_cache(maxsize=None)
def _make_sc_gather():
    info = plsc.get_sparse_core_info()
    nc, ns, nl = info.num_cores, info.num_subcores, info.num_lanes
    b_per_w = B // (nc * ns)

    def body(idx_hbm, table_hbm, out_hbm, idx_v, rows_v, sem):
        wid = lax.axis_index("s") * nc + lax.axis_index("c")
        base = wid * b_per_w
        pltpu.sync_copy(idx_hbm.at[pl.ds(base, b_per_w)], idx_v)
        descs = []
        for c in range(b_per_w // nl):
            vec = idx_v[pl.ds(c * nl, nl)]
            for j in range(nl):
                row = vec[j]
                i = c * nl + j
                d = pltpu.make_async_copy(
                    table_hbm.at[pl.ds(row, 1)], rows_v.at[pl.ds(i, 1)], sem
                )
                d.start()
                descs.append(d)
        for d in descs:
            d.wait()
        pltpu.sync_copy(rows_v, out_hbm.at[pl.ds(base, b_per_w)])

    return pl.kernel(
        body,
        out_type=jax.ShapeDtypeStruct((B, DIM), jnp.float32),
        mesh=plsc.VectorSubcoreMesh(core_axis_name="c", subcore_axis_name="s"),
        scratch_types=[
            pltpu.VMEM((b_per_w,), jnp.int32),
            pltpu.VMEM((b_per_w, DIM), jnp.float32),
            pltpu.SemaphoreType.DMA,
        ],
    )


V_BLK = 4096


def _mm_body(e_ref, wt_ref, b_ref, o_ref, ent_ref):
    @pl.when(pl.program_id(0) == 0)
    def _():
        e = e_ref[...]
        ss = jnp.sum(e * e, axis=1, keepdims=True)
        norm = jnp.sqrt(ss)
        scale = jnp.where(norm > MAX_NORM, MAX_NORM / (norm + 1e-7), 1.0)
        ent_ref[...] = (e * scale).astype(jnp.bfloat16).T

    o_ref[...] = lax.dot_general(
        wt_ref[...].astype(jnp.bfloat16), ent_ref[...],
        (((0,), (0,)), ((), ())),
        preferred_element_type=jnp.float32,
    ) + b_ref[...].reshape(V_BLK, 1)


def _tc_matmul(e, Wt, b1):
    nblk = pl.cdiv(VOCAB, V_BLK)
    return pl.pallas_call(
        _mm_body,
        grid=(nblk,),
        in_specs=[
            pl.BlockSpec((B, DIM), lambda i: (0, 0)),
            pl.BlockSpec((DIM, V_BLK), lambda i: (0, i)),
            pl.BlockSpec((V_BLK,), lambda i: (i,)),
        ],
        out_specs=pl.BlockSpec((V_BLK, B), lambda i: (i, 0)),
        out_shape=jax.ShapeDtypeStruct((VOCAB, B), jnp.float32),
        scratch_shapes=[pltpu.VMEM((DIM, B), jnp.bfloat16)],
    )(e, Wt, b1)


def kernel(inputs, emb, W, b):
    idx = inputs.astype(jnp.int32)
    e = _make_sc_gather()(idx, emb)
    out_t = _tc_matmul(e, W.T, b)
    return out_t.T
